# Initial kernel scaffold; baseline (speedup 1.0000x reference)
import functools

import jax
import jax.numpy as jnp
from jax.experimental import pallas as pl
from jax.experimental.pallas import tpu as pltpu

N, C_IN, H, W_ = 8, 384, 32, 32
HD, FC, CS, FS = 384, 8, 8, 2
C2 = 2 * HD
SH = H // FS
SW = W_ // FS
L = SH * SW
S = CS * CS
SC2 = C2 // FC
SCH = SC2 // 2
NFOLD = N * FS * FS


def _fold_kernel(a_ref, wp_ref, bp_ref, wm_ref, bm_ref, ab_ref, out_ref):
    a = a_ref[0]
    alpha = ab_ref[0]
    beta = ab_ref[1]
    xt = jnp.dot(a, wp_ref[...].T, preferred_element_type=jnp.float32)
    xt = xt + bp_ref[...]
    r_i = jax.lax.broadcasted_iota(jnp.int32, (S, L), 0)
    l_i = jax.lax.broadcasted_iota(jnp.int32, (S, L), 1)
    sel = ((l_i // SW) // (SH // CS) == r_i // CS) & (
        (l_i % SW) // (SW // CS) == r_i % CS)
    P = jnp.where(sel, 1.0 / ((SH // CS) * (SW // CS)), 0.0)
    ct = jnp.dot(P, xt, preferred_element_type=jnp.float32)

    disp_heads = []
    for h in range(FC):
        base = h * SC2
        xp = xt[:, base:base + SCH]
        xv = xt[:, base + SCH:base + SC2]
        cp = ct[:, base:base + SCH]
        cv = ct[:, base + SCH:base + SC2]
        xn = xp / jnp.maximum(
            jnp.sqrt(jnp.sum(xp * xp, axis=1, keepdims=True)), 1e-12)
        cn = cp / jnp.maximum(
            jnp.sqrt(jnp.sum(cp * cp, axis=1, keepdims=True)), 1e-12)
        sim = jnp.dot(xn, cn.T, preferred_element_type=jnp.float32)
        sim = jax.nn.sigmoid(alpha * sim + beta)
        idx = jnp.argmax(sim, axis=1)
        s_i = jax.lax.broadcasted_iota(jnp.int32, (L, S), 1)
        weighted = jnp.where(s_i == idx[:, None], sim, 0.0)
        aggv = jax.lax.dot_general(
            weighted, xv, (((0,), (0,)), ((), ())),
            preferred_element_type=jnp.float32)
        den = 1.0 + jnp.sum(weighted, axis=0)
        agg_n = (cv + aggv) / den[:, None]
        disp_heads.append(
            jnp.dot(weighted, agg_n, preferred_element_type=jnp.float32))
    disp = jnp.concatenate(disp_heads, axis=1)
    out = jnp.dot(disp, wm_ref[...].T, preferred_element_type=jnp.float32)
    out_ref[0] = out + bm_ref[...]


@functools.partial(jax.jit, static_argnames=("interpret",))
def kernel(x, W_proj, b_proj, W_merge, b_merge, alpha, beta, interpret=False):
    a = x.reshape(N, C_IN, FS, SH, FS, SW).transpose(0, 2, 4, 3, 5, 1)
    a = a.reshape(NFOLD, L, C_IN)
    ab = jnp.concatenate([alpha, beta]).astype(jnp.float32)
    out = pl.pallas_call(
        _fold_kernel,
        grid=(NFOLD,),
        in_specs=[
            pl.BlockSpec((1, L, C_IN), lambda f: (f, 0, 0)),
            pl.BlockSpec((C2, C_IN), lambda f: (0, 0)),
            pl.BlockSpec((C2,), lambda f: (0,)),
            pl.BlockSpec((C_IN, HD), lambda f: (0, 0)),
            pl.BlockSpec((C_IN,), lambda f: (0,)),
            pl.BlockSpec(memory_space=pltpu.SMEM),
        ],
        out_specs=pl.BlockSpec((1, L, C_IN), lambda f: (f, 0, 0)),
        out_shape=jax.ShapeDtypeStruct((NFOLD, L, C_IN), jnp.float32),
        interpret=interpret,
    )(a, W_proj, b_proj, W_merge, b_merge, ab)
    out = out.reshape(N, FS, FS, SH, SW, C_IN).transpose(0, 5, 1, 3, 2, 4)
    return out.reshape(N, C_IN, H, W_)


# speedup vs baseline: 3.0368x; 3.0368x over previous
import functools

import jax
import jax.numpy as jnp
from jax.experimental import pallas as pl
from jax.experimental.pallas import tpu as pltpu

N, C_IN, H, W_ = 8, 384, 32, 32
HD, FC, CS, FS = 384, 8, 8, 2
C2 = 2 * HD
SH = H // FS
SW = W_ // FS
L = SH * SW
S = CS * CS
SC2 = C2 // FC
SCH = SC2 // 2
NFOLD = N * FS * FS


def _fold_kernel(a_ref, wp_ref, bp_ref, wm_ref, bm_ref, ab_ref, out_ref):
    a = a_ref[0]
    alpha = ab_ref[0]
    beta = ab_ref[1]
    xt = jnp.dot(a, wp_ref[...].T, preferred_element_type=jnp.float32,
                 precision=jax.lax.Precision.DEFAULT)
    xt = xt + bp_ref[...]
    r_i = jax.lax.broadcasted_iota(jnp.int32, (S, L), 0)
    l_i = jax.lax.broadcasted_iota(jnp.int32, (S, L), 1)
    sel = ((l_i // SW) // (SH // CS) == r_i // CS) & (
        (l_i % SW) // (SW // CS) == r_i % CS)
    P = jnp.where(sel, 1.0 / ((SH // CS) * (SW // CS)), 0.0)
    ct = jnp.dot(P, xt, preferred_element_type=jnp.float32, precision=jax.lax.Precision.HIGHEST)

    disp_heads = []
    for h in range(FC):
        base = h * SC2
        xp = xt[:, base:base + SCH]
        xv = xt[:, base + SCH:base + SC2]
        cp = ct[:, base:base + SCH]
        cv = ct[:, base + SCH:base + SC2]
        xn = xp / jnp.maximum(
            jnp.sqrt(jnp.sum(xp * xp, axis=1, keepdims=True)), 1e-12)
        cn = cp / jnp.maximum(
            jnp.sqrt(jnp.sum(cp * cp, axis=1, keepdims=True)), 1e-12)
        sim = jnp.dot(xn, cn.T, preferred_element_type=jnp.float32,
                      precision=jax.lax.Precision.DEFAULT)
        sim = jax.nn.sigmoid(alpha * sim + beta)
        idx = jnp.argmax(sim, axis=1)
        s_i = jax.lax.broadcasted_iota(jnp.int32, (L, S), 1)
        weighted = jnp.where(s_i == idx[:, None], sim, 0.0)
        aggv = jax.lax.dot_general(
            weighted, xv, (((0,), (0,)), ((), ())),
            preferred_element_type=jnp.float32, precision=jax.lax.Precision.HIGHEST)
        den = 1.0 + jnp.sum(weighted, axis=0)
        agg_n = (cv + aggv) / den[:, None]
        disp_heads.append(
            jnp.dot(weighted, agg_n, preferred_element_type=jnp.float32, precision=jax.lax.Precision.HIGHEST))
    disp = jnp.concatenate(disp_heads, axis=1)
    out = jnp.dot(disp, wm_ref[...].T, preferred_element_type=jnp.float32, precision=jax.lax.Precision.HIGHEST)
    out_ref[0] = out + bm_ref[...]


@functools.partial(jax.jit, static_argnames=("interpret",))
def kernel(x, W_proj, b_proj, W_merge, b_merge, alpha, beta, interpret=False):
    a = x.reshape(N, C_IN, FS, SH, FS, SW).transpose(0, 2, 4, 3, 5, 1)
    a = a.reshape(NFOLD, L, C_IN)
    ab = jnp.concatenate([alpha, beta]).astype(jnp.float32)
    out = pl.pallas_call(
        _fold_kernel,
        grid=(NFOLD,),
        in_specs=[
            pl.BlockSpec((1, L, C_IN), lambda f: (f, 0, 0)),
            pl.BlockSpec((C2, C_IN), lambda f: (0, 0)),
            pl.BlockSpec((C2,), lambda f: (0,)),
            pl.BlockSpec((C_IN, HD), lambda f: (0, 0)),
            pl.BlockSpec((C_IN,), lambda f: (0,)),
            pl.BlockSpec(memory_space=pltpu.SMEM),
        ],
        out_specs=pl.BlockSpec((1, L, C_IN), lambda f: (f, 0, 0)),
        out_shape=jax.ShapeDtypeStruct((NFOLD, L, C_IN), jnp.float32),
        interpret=interpret,
    )(a, W_proj, b_proj, W_merge, b_merge, ab)
    out = out.reshape(N, FS, FS, SH, SW, C_IN).transpose(0, 5, 1, 3, 2, 4)
    return out.reshape(N, C_IN, H, W_)
